# baseline (speedup 1.0000x reference)
import functools

import jax
import jax.numpy as jnp
from jax.experimental import pallas as pl
from jax.experimental.pallas import tpu as pltpu

_EPS = 1e-5
_VMEM_LIMIT = 48 * 1024 * 1024


def _pick_tile(b, pref):
    t = min(pref, b)
    while t > 8 and b % t:
        t //= 2
    return max(t, 1)


def _fused_kernel(x_ref, w_ref, gamma_ref, beta_ref, out_ref,
                  hc_ref, wb_ref, sum_ref, ssq_ref, scale_ref, shift_ref,
                  *, inv_n, nb):
    p = pl.program_id(0)
    i = pl.program_id(1)

    @pl.when(p == 0)
    def _stats():
        @pl.when(i == 0)
        def _init():
            wb_ref[...] = w_ref[...].astype(jnp.bfloat16)
            sum_ref[...] = jnp.zeros_like(sum_ref)
            ssq_ref[...] = jnp.zeros_like(ssq_ref)

        x_t = x_ref[...]
        h = jax.lax.dot_general(
            x_t.astype(jnp.bfloat16), wb_ref[...],
            dimension_numbers=(((1,), (1,)), ((), ())),
            preferred_element_type=jnp.float32)

        sum_ref[...] += jnp.sum(h, axis=0, keepdims=True)
        ssq_ref[...] += jnp.sum(h * h, axis=0, keepdims=True)
        hc_ref[i] = h.astype(jnp.bfloat16)
        out_ref[...] = x_t

        @pl.when(i == nb - 1)
        def _finalize():
            mean = sum_ref[...] * inv_n
            var = jnp.maximum(ssq_ref[...] * inv_n - mean * mean, 0.0)
            scale = gamma_ref[...] * jax.lax.rsqrt(var + _EPS)
            scale_ref[...] = scale
            shift_ref[...] = beta_ref[...] - mean * scale

    @pl.when(p == 1)
    def _apply():
        h = hc_ref[i].astype(jnp.float32)
        out_ref[...] = jnp.maximum(h * scale_ref[...] + shift_ref[...], 0.0)


@jax.jit
def _run(x, w, gamma, beta):
    f32 = jnp.float32
    B, I = x.shape
    O = w.shape[0]
    x = x.astype(f32)
    w = w.astype(f32)
    g2 = gamma.astype(f32).reshape(1, O)
    b2 = beta.astype(f32).reshape(1, O)

    tb = _pick_tile(B, 512)
    nb = B // tb

    out = pl.pallas_call(
        functools.partial(_fused_kernel, inv_n=1.0 / B, nb=nb),
        grid=(2, nb),
        in_specs=[
            pl.BlockSpec((tb, I), lambda p, i: ((1 - p) * i + p * (nb - 1), 0)),
            pl.BlockSpec((O, I), lambda p, i: (0, 0)),
            pl.BlockSpec((1, O), lambda p, i: (0, 0)),
            pl.BlockSpec((1, O), lambda p, i: (0, 0)),
        ],
        out_specs=pl.BlockSpec((tb, O), lambda p, i: (i, 1 - p)),
        out_shape=jax.ShapeDtypeStruct((B, O + I), f32),
        scratch_shapes=[
            pltpu.VMEM((nb, tb, O), jnp.bfloat16),
            pltpu.VMEM((O, I), jnp.bfloat16),
            pltpu.VMEM((1, O), f32),
            pltpu.VMEM((1, O), f32),
            pltpu.VMEM((1, O), f32),
            pltpu.VMEM((1, O), f32),
        ],
        compiler_params=pltpu.CompilerParams(
            dimension_semantics=("arbitrary", "arbitrary"),
            vmem_limit_bytes=_VMEM_LIMIT,
        ),
    )(x, w, g2, b2)
    return out


def kernel(x, w, b, gamma, beta):
    del b
    return _run(x, w, gamma, beta)


# speedup vs baseline: 1.1185x; 1.1185x over previous
import functools

import jax
import jax.numpy as jnp
from jax.experimental import pallas as pl
from jax.experimental.pallas import tpu as pltpu

_EPS = 1e-5
_VMEM_LIMIT = 60000 * 1024


def _pick_tile(b, pref):
    t = min(pref, b)
    while t > 8 and b % t:
        t //= 2
    return max(t, 1)


def _fused_kernel(x_ref, w_ref, gamma_ref, beta_ref, out_ref,
                  hc_ref, wb_ref, sum_ref, ssq_ref, scale_ref, shift_ref,
                  *, inv_n, nb, chunk):
    p = pl.program_id(0)
    i = pl.program_id(1)
    tb = x_ref.shape[0]
    slices = [pl.ds(k, chunk) for k in range(0, tb, chunk)]

    def _matmul(x_chunk):
        return jax.lax.dot_general(
            x_chunk.astype(jnp.bfloat16), wb_ref[...],
            dimension_numbers=(((1,), (1,)), ((), ())),
            preferred_element_type=jnp.float32)

    @pl.when(p == 0)
    def _stats():
        @pl.when(i == 0)
        def _init():
            wb_ref[...] = w_ref[...].astype(jnp.bfloat16)
            sum_ref[...] = jnp.zeros_like(sum_ref)
            ssq_ref[...] = jnp.zeros_like(ssq_ref)

        for sl in slices:
            h = _matmul(x_ref[sl, :])
            sum_ref[...] += jnp.sum(h, axis=0, keepdims=True)
            ssq_ref[...] += jnp.sum(h * h, axis=0, keepdims=True)

            @pl.when(i < nb - 1)
            def _cache():
                hc_ref[i, sl, :] = h.astype(jnp.bfloat16)

            out_ref[sl, :] = x_ref[sl, :]

        @pl.when(i == nb - 1)
        def _finalize():
            mean = sum_ref[...] * inv_n
            var = jnp.maximum(ssq_ref[...] * inv_n - mean * mean, 0.0)
            scale = gamma_ref[...] * jax.lax.rsqrt(var + _EPS)
            scale_ref[...] = scale
            shift_ref[...] = beta_ref[...] - mean * scale

    @pl.when(p == 1)
    def _apply():
        for sl in slices:
            @pl.when(i < nb - 1)
            def _from_cache():
                h = hc_ref[i, sl, :].astype(jnp.float32)
                out_ref[sl, :] = jnp.maximum(
                    h * scale_ref[...] + shift_ref[...], 0.0)

            @pl.when(i == nb - 1)
            def _recompute():
                h = _matmul(x_ref[sl, :])
                out_ref[sl, :] = jnp.maximum(
                    h * scale_ref[...] + shift_ref[...], 0.0)


@jax.jit
def _run(x, w, gamma, beta):
    f32 = jnp.float32
    B, I = x.shape
    O = w.shape[0]
    x = x.astype(f32)
    w = w.astype(f32)
    g2 = gamma.astype(f32).reshape(1, O)
    b2 = beta.astype(f32).reshape(1, O)

    tb = _pick_tile(B, 2048)
    nb = B // tb

    out = pl.pallas_call(
        functools.partial(_fused_kernel, inv_n=1.0 / B, nb=nb,
                          chunk=_pick_tile(tb, 1024)),
        grid=(2, nb),
        in_specs=[
            pl.BlockSpec((tb, I), lambda p, i: ((1 - p) * i + p * (nb - 1), 0)),
            pl.BlockSpec((O, I), lambda p, i: (0, 0)),
            pl.BlockSpec((1, O), lambda p, i: (0, 0)),
            pl.BlockSpec((1, O), lambda p, i: (0, 0)),
        ],
        out_specs=pl.BlockSpec((tb, O), lambda p, i: (i, 1 - p)),
        out_shape=jax.ShapeDtypeStruct((B, O + I), f32),
        scratch_shapes=[
            pltpu.VMEM((max(nb - 1, 1), tb, O), jnp.bfloat16),
            pltpu.VMEM((O, I), jnp.bfloat16),
            pltpu.VMEM((1, O), f32),
            pltpu.VMEM((1, O), f32),
            pltpu.VMEM((1, O), f32),
            pltpu.VMEM((1, O), f32),
        ],
        compiler_params=pltpu.CompilerParams(
            dimension_semantics=("arbitrary", "arbitrary"),
            vmem_limit_bytes=_VMEM_LIMIT,
        ),
    )(x, w, g2, b2)
    return out


def kernel(x, w, b, gamma, beta):
    del b
    return _run(x, w, gamma, beta)


# speedup vs baseline: 1.1414x; 1.0205x over previous
import functools

import jax
import jax.numpy as jnp
from jax.experimental import pallas as pl
from jax.experimental.pallas import tpu as pltpu

_EPS = 1e-5
_VMEM_LIMIT = 60000 * 1024


def _pick_tile(b, pref):
    t = min(pref, b)
    while t > 8 and b % t:
        t //= 2
    return max(t, 1)


def _fused_kernel(x_ref, w_ref, gamma_ref, beta_ref, out_ref,
                  hc_ref, wb_ref, sum_ref, ssq_ref, scale_ref, shift_ref,
                  *, inv_n, nb, chunk):
    p = pl.program_id(0)
    i = pl.program_id(1)
    tb = x_ref.shape[0]
    slices = [pl.ds(k, chunk) for k in range(0, tb, chunk)]

    def _matmul(x_chunk):
        return jax.lax.dot_general(
            x_chunk.astype(jnp.bfloat16), wb_ref[...],
            dimension_numbers=(((1,), (1,)), ((), ())),
            preferred_element_type=jnp.float32)

    @pl.when(p == 0)
    def _stats():
        @pl.when(i == 0)
        def _init():
            wb_ref[...] = w_ref[...].astype(jnp.bfloat16)
            sum_ref[...] = jnp.zeros_like(sum_ref)
            ssq_ref[...] = jnp.zeros_like(ssq_ref)

        for sl in slices:
            h = _matmul(x_ref[sl, :])
            sum_ref[...] += jnp.sum(h, axis=0, keepdims=True)
            ssq_ref[...] += jnp.sum(h * h, axis=0, keepdims=True)

            @pl.when(i < nb - 1)
            def _cache():
                hc_ref[i, sl, :] = h.astype(jnp.bfloat16)

            out_ref[sl, :] = x_ref[sl, :]

        @pl.when(i == nb - 1)
        def _finalize():
            mean = sum_ref[...] * inv_n
            var = jnp.maximum(ssq_ref[...] * inv_n - mean * mean, 0.0)
            scale = gamma_ref[...] * jax.lax.rsqrt(var + _EPS)
            scale_ref[...] = scale
            shift_ref[...] = beta_ref[...] - mean * scale

    @pl.when(p == 1)
    def _apply():
        for sl in slices:
            @pl.when(i < nb - 1)
            def _from_cache():
                h = hc_ref[i, sl, :].astype(jnp.float32)
                out_ref[sl, :] = jnp.maximum(
                    h * scale_ref[...] + shift_ref[...], 0.0)

            @pl.when(i == nb - 1)
            def _recompute():
                h = _matmul(x_ref[sl, :])
                out_ref[sl, :] = jnp.maximum(
                    h * scale_ref[...] + shift_ref[...], 0.0)


@jax.jit
def _run(x, w, gamma, beta):
    f32 = jnp.float32
    B, I = x.shape
    O = w.shape[0]
    x = x.astype(f32)
    w = w.astype(f32)
    g2 = gamma.astype(f32).reshape(1, O)
    b2 = beta.astype(f32).reshape(1, O)

    tb = _pick_tile(B, 1024)
    nb = B // tb

    out = pl.pallas_call(
        functools.partial(_fused_kernel, inv_n=1.0 / B, nb=nb,
                          chunk=_pick_tile(tb, 1024)),
        grid=(2, nb),
        in_specs=[
            pl.BlockSpec((tb, I), lambda p, i: ((1 - p) * i + p * (nb - 1), 0)),
            pl.BlockSpec((O, I), lambda p, i: (0, 0)),
            pl.BlockSpec((1, O), lambda p, i: (0, 0)),
            pl.BlockSpec((1, O), lambda p, i: (0, 0)),
        ],
        out_specs=pl.BlockSpec((tb, O), lambda p, i: (i, 1 - p)),
        out_shape=jax.ShapeDtypeStruct((B, O + I), f32),
        scratch_shapes=[
            pltpu.VMEM((max(nb - 1, 1), tb, O), jnp.bfloat16),
            pltpu.VMEM((O, I), jnp.bfloat16),
            pltpu.VMEM((1, O), f32),
            pltpu.VMEM((1, O), f32),
            pltpu.VMEM((1, O), f32),
            pltpu.VMEM((1, O), f32),
        ],
        compiler_params=pltpu.CompilerParams(
            dimension_semantics=("arbitrary", "arbitrary"),
            vmem_limit_bytes=_VMEM_LIMIT,
        ),
    )(x, w, g2, b2)
    return out


def kernel(x, w, b, gamma, beta):
    del b
    return _run(x, w, gamma, beta)


# speedup vs baseline: 1.2294x; 1.0771x over previous
import functools

import jax
import jax.numpy as jnp
from jax.experimental import pallas as pl
from jax.experimental.pallas import tpu as pltpu

_EPS = 1e-5
_VMEM_LIMIT = 60000 * 1024


def _pick_tile(b, pref):
    t = min(pref, b)
    while t > 8 and b % t:
        t //= 2
    return max(t, 1)


def _fused_kernel(x_ref, w_ref, gamma_ref, beta_ref, out_ref,
                  hc_ref, wb_ref, sum_ref, ssq_ref, scale_ref, shift_ref,
                  *, inv_n, nb, chunk):
    p = pl.program_id(0)
    i = pl.program_id(1)
    tb = x_ref.shape[0]
    slices = [pl.ds(k, chunk) for k in range(0, tb, chunk)]

    def _matmul(x_chunk):
        return jax.lax.dot_general(
            x_chunk.astype(jnp.bfloat16), wb_ref[...],
            dimension_numbers=(((1,), (1,)), ((), ())),
            preferred_element_type=jnp.float32)

    @pl.when(p == 0)
    def _stats():
        @pl.when(i == 0)
        def _init():
            wb_ref[...] = w_ref[...].astype(jnp.bfloat16)
            sum_ref[...] = jnp.zeros_like(sum_ref)
            ssq_ref[...] = jnp.zeros_like(ssq_ref)

        for sl in slices:
            h = _matmul(x_ref[sl, :])
            sum_ref[...] += jnp.sum(h, axis=0, keepdims=True)
            ssq_ref[...] += jnp.sum(h * h, axis=0, keepdims=True)
            hc_ref[i, sl, :] = h.astype(jnp.bfloat16)
            out_ref[sl, :] = x_ref[sl, :]

        @pl.when(i == nb - 1)
        def _finalize():
            mean = sum_ref[...] * inv_n
            var = jnp.maximum(ssq_ref[...] * inv_n - mean * mean, 0.0)
            scale = gamma_ref[...] * jax.lax.rsqrt(var + _EPS)
            scale_ref[...] = scale
            shift_ref[...] = beta_ref[...] - mean * scale

    @pl.when(p == 1)
    def _apply():
        for sl in slices:
            h = hc_ref[i, sl, :].astype(jnp.float32)
            out_ref[sl, :] = jnp.maximum(
                h * scale_ref[...] + shift_ref[...], 0.0)


@jax.jit
def _run(x, w, gamma, beta):
    f32 = jnp.float32
    B, I = x.shape
    O = w.shape[0]
    x = x.astype(f32)
    w = w.astype(f32)
    g2 = gamma.astype(f32).reshape(1, O)
    b2 = beta.astype(f32).reshape(1, O)

    tb = _pick_tile(B, 1024)
    nb = B // tb

    out = pl.pallas_call(
        functools.partial(_fused_kernel, inv_n=1.0 / B, nb=nb,
                          chunk=_pick_tile(tb, 1024)),
        grid=(2, nb),
        in_specs=[
            pl.BlockSpec((tb, I), lambda p, i: ((1 - p) * i + p * (nb - 1), 0)),
            pl.BlockSpec((O, I), lambda p, i: (0, 0)),
            pl.BlockSpec((1, O), lambda p, i: (0, 0)),
            pl.BlockSpec((1, O), lambda p, i: (0, 0)),
        ],
        out_specs=pl.BlockSpec((tb, O), lambda p, i: (i, 1 - p)),
        out_shape=jax.ShapeDtypeStruct((B, O + I), f32),
        scratch_shapes=[
            pltpu.VMEM((nb, tb, O), jnp.bfloat16),
            pltpu.VMEM((O, I), jnp.bfloat16),
            pltpu.VMEM((1, O), f32),
            pltpu.VMEM((1, O), f32),
            pltpu.VMEM((1, O), f32),
            pltpu.VMEM((1, O), f32),
        ],
        compiler_params=pltpu.CompilerParams(
            dimension_semantics=("arbitrary", "arbitrary"),
            vmem_limit_bytes=_VMEM_LIMIT,
        ),
    )(x, w, g2, b2)
    return out


def kernel(x, w, b, gamma, beta):
    del b
    return _run(x, w, gamma, beta)
